# baseline (speedup 1.0000x reference)
import jax
import jax.numpy as jnp
from jax.experimental import pallas as pl
from jax.experimental.pallas import tpu as pltpu

_MARGIN = 1.0
_EPS = 1e-6
_BLK = 2048


def _triplet_loss_kernel(a_hbm, p_hbm, n_hbm, out_ref,
                         a_v, p_v, n_v, sem_a, sem_p, sem_n):
    C, D = n_hbm.shape
    n_blocks = C // _BLK

    cp_a = pltpu.make_async_copy(a_hbm, a_v, sem_a)
    cp_a.start()
    cp_n = [
        pltpu.make_async_copy(
            n_hbm.at[pl.ds(b * _BLK, _BLK), :],
            n_v.at[pl.ds(b * _BLK, _BLK), :],
            sem_n.at[b],
        )
        for b in range(n_blocks)
    ]
    for c in cp_n:
        c.start()
    cp_p = pltpu.make_async_copy(p_hbm, p_v, sem_p)
    cp_p.start()

    cp_a.wait()
    A = a_v[:, :]
    R = A.shape[0]
    a_term = jnp.sum(A * A + (2.0 * _EPS) * A, axis=1, keepdims=True)
    ones_r = jnp.ones((R, 1), dtype=jnp.float32)
    a_aug = jnp.concatenate([A, ones_r, a_term], axis=1).astype(jnp.bfloat16)

    ones_b = jnp.ones((_BLK, 1), dtype=jnp.float32)
    best8 = jnp.full((8, R), jnp.inf, dtype=jnp.float32)
    for b in range(n_blocks):
        cp_n[b].wait()
        Nb = n_v[pl.ds(b * _BLK, _BLK), :]
        n_term = jnp.sum(Nb * Nb - (2.0 * _EPS) * Nb, axis=1, keepdims=True)
        nb = jnp.concatenate([Nb * -2.0, n_term, ones_b],
                             axis=1).astype(jnp.bfloat16)
        z = jax.lax.dot_general(nb, a_aug, (((1,), (1,)), ((), ())),
                                preferred_element_type=jnp.float32)
        best8 = jnp.minimum(best8, jnp.min(z.reshape(_BLK // 8, 8, R), axis=0))
    best = jnp.min(best8, axis=0, keepdims=True)

    dn = jnp.sqrt(jnp.maximum(best + D * _EPS * _EPS, 0.0))
    cp_p.wait()
    diff = A - p_v[:, :] + _EPS
    ones_row = jnp.ones((1, D), dtype=jnp.float32)
    dp2 = jax.lax.dot_general(ones_row, diff * diff, (((1,), (1,)), ((), ())),
                              preferred_element_type=jnp.float32)
    losses = jnp.maximum(jnp.sqrt(dp2) - dn + _MARGIN, 0.0)
    out_ref[:, :] = jnp.sum(losses, axis=1, keepdims=True) / R


def kernel(anchor, positive, negative):
    out = pl.pallas_call(
        _triplet_loss_kernel,
        in_specs=[pl.BlockSpec(memory_space=pltpu.MemorySpace.HBM)] * 3,
        out_shape=jax.ShapeDtypeStruct((1, 1), jnp.float32),
        scratch_shapes=[
            pltpu.VMEM(anchor.shape, jnp.float32),
            pltpu.VMEM(positive.shape, jnp.float32),
            pltpu.VMEM(negative.shape, jnp.float32),
            pltpu.SemaphoreType.DMA,
            pltpu.SemaphoreType.DMA,
            pltpu.SemaphoreType.DMA((negative.shape[0] // _BLK,)),
        ],
    )(anchor, positive, negative)
    return out[0, 0]


# speedup vs baseline: 1.0306x; 1.0306x over previous
import jax
import jax.numpy as jnp
from jax.experimental import pallas as pl
from jax.experimental.pallas import tpu as pltpu

_MARGIN = 1.0
_EPS = 1e-6
_BLK = 2048


def _triplet_loss_kernel(a_hbm, p_hbm, n_hbm, out_ref,
                         a_v, p_v, n_v, sem_a, sem_p, sem_n):
    cp_a = pltpu.make_async_copy(a_hbm, a_v, sem_a)
    cp_p = pltpu.make_async_copy(p_hbm, p_v, sem_p)
    cp_n = pltpu.make_async_copy(n_hbm, n_v, sem_n)
    cp_a.start()
    cp_n.start()
    cp_p.start()
    cp_a.wait()

    A = a_v[:, :]
    R, D = A.shape

    a_term = jnp.sum(A * A + (2.0 * _EPS) * A, axis=1, keepdims=True)
    ones_r = jnp.ones((R, 1), dtype=jnp.float32)
    a_aug = jnp.concatenate([A, ones_r, a_term], axis=1).astype(jnp.bfloat16)

    cp_n.wait()
    N = n_v[:, :]
    C = N.shape[0]
    n_term = jnp.sum(N * N - (2.0 * _EPS) * N, axis=1, keepdims=True)
    ones_c = jnp.ones((C, 1), dtype=jnp.float32)
    n_aug = jnp.concatenate([N * -2.0, n_term, ones_c],
                            axis=1).astype(jnp.bfloat16)

    best8 = jnp.full((8, R), jnp.inf, dtype=jnp.float32)
    for b in range(C // _BLK):
        nb = jax.lax.slice(n_aug, (b * _BLK, 0), ((b + 1) * _BLK, D + 2))
        z = jax.lax.dot_general(nb, a_aug, (((1,), (1,)), ((), ())),
                                preferred_element_type=jnp.float32)
        best8 = jnp.minimum(best8, jnp.min(z.reshape(_BLK // 8, 8, R), axis=0))
    best = jnp.min(best8, axis=0, keepdims=True)

    dn = jnp.sqrt(jnp.maximum(best + D * _EPS * _EPS, 0.0))
    cp_p.wait()
    diff = A - p_v[:, :] + _EPS
    ones_row = jnp.ones((1, D), dtype=jnp.float32)
    dp2 = jax.lax.dot_general(ones_row, diff * diff, (((1,), (1,)), ((), ())),
                              preferred_element_type=jnp.float32)
    losses = jnp.maximum(jnp.sqrt(dp2) - dn + _MARGIN, 0.0)
    out_ref[:, :] = jnp.sum(losses, axis=1, keepdims=True) / R


def kernel(anchor, positive, negative):
    out = pl.pallas_call(
        _triplet_loss_kernel,
        in_specs=[pl.BlockSpec(memory_space=pltpu.MemorySpace.HBM)] * 3,
        out_shape=jax.ShapeDtypeStruct((1, 1), jnp.float32),
        scratch_shapes=[
            pltpu.VMEM(anchor.shape, jnp.float32),
            pltpu.VMEM(positive.shape, jnp.float32),
            pltpu.VMEM(negative.shape, jnp.float32),
            pltpu.SemaphoreType.DMA,
            pltpu.SemaphoreType.DMA,
            pltpu.SemaphoreType.DMA,
        ],
    )(anchor, positive, negative)
    return out[0, 0]
